# baseline (speedup 1.0000x reference)
import functools

import jax
import jax.numpy as jnp
from jax import lax
from jax.experimental import pallas as pl
from jax.experimental.pallas import tpu as pltpu
from jax.experimental.pallas import tpu_sc as plsc

_N_BEH = 3
_N_EDGES = 320000
_N_NODES = 10000
_D = 128
_CHUNK = 128
_N_TILES = 16
_EPT = _N_EDGES // _N_TILES
_NCH = 160
_PADE = _NCH * _CHUNK - _EPT
_RPT = 624
_TAIL = _N_NODES - _RPT * _N_TILES
_SEC = 40
_NSEC = _NCH // _SEC


def _sc_aggregate(emb, sidx5, didx5, val4):
    mesh = plsc.VectorSubcoreMesh(core_axis_name="c", subcore_axis_name="s")

    @functools.partial(
        pl.kernel,
        out_type=jax.ShapeDtypeStruct((2, _N_BEH, _N_NODES, _D), jnp.float32),
        mesh=mesh,
        scratch_types=[
            pltpu.VMEM_SHARED((_N_NODES, _D), jnp.float32),
            pltpu.VMEM((_SEC, _CHUNK), jnp.int32),
            pltpu.VMEM((_SEC, _CHUNK), jnp.int32),
            pltpu.VMEM((_SEC, _CHUNK), jnp.float32),
            pltpu.VMEM((_CHUNK, _D), jnp.float32),
            pltpu.VMEM((_CHUNK, _D), jnp.float32),
            pltpu.SemaphoreType.DMA,
            pltpu.SemaphoreType.DMA,
        ],
    )
    def agg_kernel(emb_hbm, sidx_hbm, didx_hbm, val_hbm, out_hbm,
                   acc, sidx_all, didx_all, val_all, rows0, rows1,
                   sem_g0, sem_g1):
        cid = lax.axis_index("c")
        sid = lax.axis_index("s")

        zeros_f = jnp.zeros((16,), jnp.float32)
        rows = (rows0, rows1)
        sem_g = (sem_g0, sem_g1)

        def zrow(e, _):
            for k in range(_D // 16):
                rows0[e, pl.ds(k * 16, 16)] = zeros_f
            return 0

        def scale(rows_ref, j):
            def grp(g, _):
                vv = val_all[j, pl.ds(g * 16, 16)]
                for el in range(16):
                    v = jnp.full((16,), vv[el], jnp.float32)
                    e = g * 16 + el
                    for k in range(_D // 16):
                        sl = pl.ds(k * 16, 16)
                        rows_ref[e, sl] = rows_ref[e, sl] * v
                return 0
            lax.fori_loop(0, _CHUNK // 16, grp, 0)

        for b in range(_N_BEH):
            lax.fori_loop(0, _CHUNK, zrow, 0)
            row0 = sid * _RPT
            for z, n in ((0, _CHUNK), (1, _CHUNK), (2, _CHUNK), (3, _CHUNK),
                         (4, _RPT - 4 * _CHUNK)):
                pltpu.sync_copy(rows0.at[pl.ds(0, n)],
                                acc.at[pl.ds(row0 + z * _CHUNK, n)])

            @pl.when(sid == 0)
            def _():
                pltpu.sync_copy(rows0.at[pl.ds(0, _TAIL)],
                                acc.at[pl.ds(_N_TILES * _RPT, _TAIL)])

            plsc.subcore_barrier()

            for sec in range(_NSEC):
                sc0 = sec * _SEC
                pltpu.sync_copy(
                    sidx_hbm.at[cid, b, sid, pl.ds(sc0, _SEC)], sidx_all)
                pltpu.sync_copy(
                    didx_hbm.at[cid, b, sid, pl.ds(sc0, _SEC)], didx_all)
                pltpu.sync_copy(val_hbm.at[b, sid, pl.ds(sc0, _SEC)], val_all)

                pltpu.async_copy(emb_hbm.at[sidx_all.at[0]], rows0, sem_g0)

                def body(j2, _):
                    for p in range(2):
                        j = j2 * 2 + p
                        rc, rn = rows[p], rows[1 - p]

                        @pl.when(j + 1 < _SEC)
                        def _():
                            pltpu.async_copy(
                                emb_hbm.at[sidx_all.at[j + 1]], rn,
                                sem_g[1 - p])

                        pltpu.make_async_copy(
                            emb_hbm.at[sidx_all.at[j]], rc, sem_g[p]).wait()
                        scale(rc, j)
                        pltpu.sync_copy(rc, acc.at[didx_all.at[j]], add=True)
                    return 0

                lax.fori_loop(0, _SEC // 2, body, 0)

            plsc.subcore_barrier()

            pltpu.sync_copy(acc.at[pl.ds(row0, _RPT)],
                            out_hbm.at[cid, b, pl.ds(row0, _RPT)])

            @pl.when(sid == 0)
            def _():
                pltpu.sync_copy(
                    acc.at[pl.ds(_N_TILES * _RPT, _TAIL)],
                    out_hbm.at[cid, b, pl.ds(_N_TILES * _RPT, _TAIL)])

            plsc.subcore_barrier()

    return agg_kernel(emb, sidx5, didx5, val4)


_ROWS_BLK = 400


def _proj_body(agg_ref, w_ref, big_ref, mean_ref):
    w = w_ref[0]
    acc = None
    for b in range(_N_BEH):
        y = jnp.dot(agg_ref[0, b], w, preferred_element_type=jnp.float32)
        big_ref[0, b] = jax.nn.sigmoid(y)
        acc = y if acc is None else acc + y
    mean_ref[0] = jax.nn.sigmoid(acc * (1.0 / _N_BEH))


def _tc_project(agg, w2):
    grid = (2, _N_NODES // _ROWS_BLK)
    return pl.pallas_call(
        _proj_body,
        grid=grid,
        in_specs=[
            pl.BlockSpec((1, _N_BEH, _ROWS_BLK, _D), lambda s, r: (s, 0, r, 0)),
            pl.BlockSpec((1, _D, _D), lambda s, r: (s, 0, 0)),
        ],
        out_specs=[
            pl.BlockSpec((1, _N_BEH, _ROWS_BLK, _D), lambda s, r: (s, 0, r, 0)),
            pl.BlockSpec((1, _ROWS_BLK, _D), lambda s, r: (s, r, 0)),
        ],
        out_shape=[
            jax.ShapeDtypeStruct((2, _N_BEH, _N_NODES, _D), jnp.float32),
            jax.ShapeDtypeStruct((2, _N_NODES, _D), jnp.float32),
        ],
    )(agg, w2)


def _pad_chunks(x):
    pad = [(0, 0)] * (x.ndim - 1) + [(0, _PADE)]
    return jnp.pad(x, pad).reshape(*x.shape[:-1], _NCH, _CHUNK)


@jax.jit
def kernel(user_embedding, item_embedding, edge_val, u_w, i_w, edge_user, edge_item):
    emb = jnp.concatenate([item_embedding, user_embedding], axis=0)
    sidx = jnp.stack([edge_item, edge_user + _N_NODES], axis=0)
    didx = jnp.stack([edge_user, edge_item], axis=0)
    sidx5 = _pad_chunks(sidx.reshape(2, _N_BEH, _N_TILES, _EPT))
    didx5 = _pad_chunks(didx.reshape(2, _N_BEH, _N_TILES, _EPT))
    val4 = _pad_chunks(edge_val.reshape(_N_BEH, _N_TILES, _EPT))
    agg = _sc_aggregate(emb, sidx5, didx5, val4)
    w2 = jnp.stack([u_w, i_w], axis=0)
    big, mean = _tc_project(agg, w2)
    return (mean[0], mean[1], big[0], big[1])
